# baseline (speedup 1.0000x reference)
import functools

import jax
import jax.numpy as jnp
from jax import lax
from jax.experimental import pallas as pl
from jax.experimental.pallas import tpu as pltpu
from jax.experimental.pallas import tpu_sc as plsc

N = 10000
NP = 10240
E = 320000
D_IN = 128
D_HID = 128
D_OUT = 64

NC = 2
NS = 16
NW = NC * NS
E_T = E // NW
CH1 = 80
CH2 = 125
NBUF = 4

R = 512
GRID = NP // R

_mesh = plsc.VectorSubcoreMesh(
    core_axis_name="c", subcore_axis_name="s", num_cores=NC, num_subcores=NS)


@functools.partial(
    pl.kernel,
    out_type=jax.ShapeDtypeStruct((NW, NP), jnp.float32),
    mesh=_mesh,
    compiler_params=pltpu.CompilerParams(needs_layout_passes=False),
    scratch_types=[
        pltpu.VMEM((E_T,), jnp.int32),
        pltpu.VMEM((NP,), jnp.float32),
    ],
)
def _deg_kernel(dst_hbm, degp_hbm, idx_v, deg_v):
    c = lax.axis_index("c")
    s = lax.axis_index("s")
    wid = c * NS + s
    base = wid * E_T
    pltpu.sync_copy(dst_hbm.at[pl.ds(base, E_T)], idx_v)

    zeros16 = jnp.zeros((16,), jnp.float32)

    def zbody(i, _):
        deg_v[pl.ds(i * 16, 16)] = zeros16
        return 0

    lax.fori_loop(0, NP // 16, zbody, 0, unroll=4)

    ones16 = jnp.ones((16,), jnp.float32)

    def body(i, _):
        idx = idx_v[pl.ds(i * 16, 16)]
        plsc.addupdate_scatter(deg_v, [idx], ones16)
        return 0

    lax.fori_loop(0, E_T // 16, body, 0, unroll=4)
    pltpu.sync_copy(deg_v, degp_hbm.at[wid])


def _make_agg(D, CH):
    NCH = E_T // CH
    NFULL = (NCH // NBUF) * NBUF
    REM = NCH - NFULL
    assert REM in (0, 1)
    ZR = 80
    assert CH >= ZR and (NP // NS) % ZR == 0

    @functools.partial(
        pl.kernel,
        out_type=jax.ShapeDtypeStruct((NC * NP, D), jnp.float32),
        mesh=_mesh,
        compiler_params=pltpu.CompilerParams(use_tc_tiling_on_sc=False),
        scratch_types=[
            pltpu.VMEM((NBUF, CH), jnp.int32),
            pltpu.VMEM((NBUF, CH), jnp.int32),
            pltpu.VMEM((NBUF, CH, D), jnp.float32),
            pltpu.VMEM_SHARED((NP, D), jnp.float32),
        ] + [pltpu.SemaphoreType.DMA] * (4 * NBUF + 1),
    )
    def agg(g_hbm, src_hbm, dst_hbm, out_hbm, isrc_v, idst_v, rows_v,
            part_sh, *sems):
        gsem = sems[0:NBUF]
        ssem = sems[NBUF:2 * NBUF]
        issem = sems[2 * NBUF:3 * NBUF]
        idsem = sems[3 * NBUF:4 * NBUF]
        zsem = sems[4 * NBUF]
        c = lax.axis_index("c")
        s = lax.axis_index("s")
        wid = c * NS + s
        ibase = wid * NCH

        def issue_isrc(j, b):
            pltpu.async_copy(src_hbm.at[ibase + j], isrc_v.at[b], issem[b])

        def drain_isrc(b):
            pltpu.make_async_copy(src_hbm.at[0], isrc_v.at[b], issem[b]).wait()

        def issue_idst(j, b):
            pltpu.async_copy(dst_hbm.at[ibase + j], idst_v.at[b], idsem[b])

        def drain_idst(b):
            pltpu.make_async_copy(dst_hbm.at[0], idst_v.at[b], idsem[b]).wait()

        def issue_g(b):
            pltpu.async_copy(g_hbm.at[isrc_v.at[b]], rows_v.at[b], gsem[b])

        def drain_g(b):
            pltpu.make_async_copy(
                g_hbm.at[isrc_v.at[0]], rows_v.at[b], gsem[b]).wait()

        def issue_s(b):
            pltpu.async_copy(rows_v.at[b], part_sh.at[idst_v.at[b]], ssem[b],
                             add=True)

        def drain_s(b):
            pltpu.make_async_copy(
                rows_v.at[b], part_sh.at[idst_v.at[0]], ssem[b]).wait()

        for b in range(3):
            issue_isrc(b, b)
        for b in range(2):
            issue_idst(b, b)

        zeros16 = jnp.zeros((16,), jnp.float32)

        def zb(i, _):
            r = i // (D // 16)
            q = i % (D // 16)
            rows_v[0, r, pl.ds(q * 16, 16)] = zeros16
            return 0

        lax.fori_loop(0, ZR * (D // 16), zb, 0, unroll=4)

        rows_per = NP // NS
        nz = rows_per // ZR
        zsrc = rows_v.at[0, pl.ds(0, ZR)]
        for k in range(nz):
            pltpu.async_copy(zsrc, part_sh.at[pl.ds(s * rows_per + k * ZR, ZR)],
                             zsem)
        for k in range(nz):
            pltpu.make_async_copy(
                zsrc, part_sh.at[pl.ds(s * rows_per, ZR)], zsem).wait()
        plsc.subcore_barrier()

        drain_isrc(0)
        issue_g(0)
        drain_isrc(1)
        issue_g(1)

        def body(i, _):
            j0 = NBUF * i
            for b in range(NBUF):
                j = j0 + b
                b2 = (b + 2) % NBUF
                b3 = (b + 3) % NBUF
                drain_g(b)
                drain_idst(b)
                issue_s(b)
                if b >= 2:
                    drain_s(b2)
                else:
                    @pl.when(i > 0)
                    def _():
                        drain_s(b2)

                @pl.when(j + 2 < NCH)
                def _():
                    drain_isrc(b2)
                    issue_g(b2)
                    issue_idst(j + 2, b2)

                @pl.when(j + 3 < NCH)
                def _():
                    issue_isrc(j + 3, b3)

            return 0

        lax.fori_loop(0, NFULL // NBUF, body, 0)

        if REM:
            drain_g(0)
            drain_idst(0)
            issue_s(0)
            drain_s((NCH - 3) % NBUF)
            drain_s((NCH - 2) % NBUF)
            drain_s((NCH - 1) % NBUF)
        else:
            drain_s((NCH - 2) % NBUF)
            drain_s((NCH - 1) % NBUF)
        plsc.subcore_barrier()

        pltpu.sync_copy(part_sh.at[pl.ds(s * rows_per, rows_per)],
                        out_hbm.at[pl.ds(c * NP + s * rows_per, rows_per)])

    return agg


_agg128 = _make_agg(D_HID, CH1)
_agg64 = _make_agg(D_OUT, CH2)


def _k2a_body(x_ref, w1_ref, h1_ref):
    h1_ref[...] = jnp.dot(x_ref[...], w1_ref[...],
                          preferred_element_type=jnp.float32)


def _matmul1(x, W1):
    return pl.pallas_call(
        _k2a_body,
        grid=(GRID,),
        in_specs=[
            pl.BlockSpec((R, D_IN), lambda i: (i, 0)),
            pl.BlockSpec((D_IN, D_HID), lambda i: (0, 0)),
        ],
        out_specs=pl.BlockSpec((R, D_HID), lambda i: (i, 0)),
        out_shape=jax.ShapeDtypeStruct((NP, D_HID), jnp.float32),
    )(x, W1)


def _k2b_body(degp_ref, h1_ref, g1_ref, dinv_ref):
    deg = degp_ref[...]
    ones = jnp.ones((NW, 1), jnp.float32)
    degsum = lax.dot_general(deg, ones, (((0,), (0,)), ((), ())),
                             preferred_element_type=jnp.float32) + 1.0
    dinv = lax.rsqrt(degsum)
    g1_ref[...] = h1_ref[...] * dinv
    dinv_ref[...] = dinv


def _prescale(degp, h1):
    return pl.pallas_call(
        _k2b_body,
        grid=(GRID,),
        in_specs=[
            pl.BlockSpec((NW, R), lambda i: (0, i)),
            pl.BlockSpec((R, D_HID), lambda i: (i, 0)),
        ],
        out_specs=[
            pl.BlockSpec((R, D_HID), lambda i: (i, 0)),
            pl.BlockSpec((R, 1), lambda i: (i, 0)),
        ],
        out_shape=[
            jax.ShapeDtypeStruct((NP, D_HID), jnp.float32),
            jax.ShapeDtypeStruct((NP, 1), jnp.float32),
        ],
    )(degp, h1)


def _k45_body(p0_ref, p1_ref, g1_ref, dinv_ref, b1_ref, gamma_ref, beta_ref,
              w2_ref, g2_ref, out1_sc, sum_sc, sq_sc, ss_sc):
    i = pl.program_id(0)

    @pl.when(i < GRID)
    def _():
        out1 = (dinv_ref[...] * (p0_ref[...] + p1_ref[...] + g1_ref[...])
                + b1_ref[...])
        out1_sc[pl.ds(i * R, R), :] = out1

        @pl.when(i == 0)
        def _():
            sum_sc[...] = jnp.zeros_like(sum_sc)
            sq_sc[...] = jnp.zeros_like(sq_sc)

        rows = lax.broadcasted_iota(jnp.int32, (R, 1), 0) + i * R
        out1m = jnp.where(rows < N, out1, 0.0)
        sum_sc[...] += jnp.sum(out1m, axis=0, keepdims=True)
        sq_sc[...] += jnp.sum(out1m * out1m, axis=0, keepdims=True)

        @pl.when(i == GRID - 1)
        def _():
            mean = sum_sc[...] / N
            var = sq_sc[...] / N - mean * mean
            scale = gamma_ref[...] * lax.rsqrt(var + 1e-5)
            shift = beta_ref[...] - mean * scale
            ss_sc[...] = jnp.concatenate([scale, shift], axis=0)

    @pl.when(i >= GRID)
    def _():
        k = i - GRID
        out1 = out1_sc[pl.ds(k * R, R), :]
        a = jax.nn.relu(out1 * ss_sc[0:1, :] + ss_sc[1:2, :])
        h2 = jnp.dot(a, w2_ref[...], preferred_element_type=jnp.float32)
        g2_ref[...] = h2 * dinv_ref[...]


def _bn_layer2(p0p1, g1, dinv, b1, gamma, beta, W2):
    return pl.pallas_call(
        _k45_body,
        grid=(2 * GRID,),
        in_specs=[
            pl.BlockSpec((R, D_HID), lambda i: (jnp.minimum(i, GRID - 1), 0)),
            pl.BlockSpec((R, D_HID),
                         lambda i: (jnp.minimum(i, GRID - 1) + GRID, 0)),
            pl.BlockSpec((R, D_HID), lambda i: (jnp.minimum(i, GRID - 1), 0)),
            pl.BlockSpec((R, 1), lambda i: (i % GRID, 0)),
            pl.BlockSpec((1, D_HID), lambda i: (0, 0)),
            pl.BlockSpec((1, D_HID), lambda i: (0, 0)),
            pl.BlockSpec((1, D_HID), lambda i: (0, 0)),
            pl.BlockSpec((D_HID, D_OUT), lambda i: (0, 0)),
        ],
        out_specs=pl.BlockSpec((R, D_OUT),
                               lambda i: (jnp.maximum(i - GRID, 0), 0)),
        out_shape=jax.ShapeDtypeStruct((NP, D_OUT), jnp.float32),
        scratch_shapes=[
            pltpu.VMEM((NP, D_HID), jnp.float32),
            pltpu.VMEM((1, D_HID), jnp.float32),
            pltpu.VMEM((1, D_HID), jnp.float32),
            pltpu.VMEM((2, D_HID), jnp.float32),
        ],
    )(p0p1, p0p1, g1, dinv, b1, gamma, beta, W2)


def _k7_body(p0_ref, p1_ref, g2_ref, dinv_ref, b2_ref, out_ref):
    out_ref[...] = (dinv_ref[...] * (p0_ref[...] + p1_ref[...] + g2_ref[...])
                    + b2_ref[...])


def _final(p0p1, g2, dinv, b2):
    return pl.pallas_call(
        _k7_body,
        grid=(GRID,),
        in_specs=[
            pl.BlockSpec((R, D_OUT), lambda i: (i, 0)),
            pl.BlockSpec((R, D_OUT), lambda i: (i + GRID, 0)),
            pl.BlockSpec((R, D_OUT), lambda i: (i, 0)),
            pl.BlockSpec((R, 1), lambda i: (i, 0)),
            pl.BlockSpec((1, D_OUT), lambda i: (0, 0)),
        ],
        out_specs=pl.BlockSpec((R, D_OUT), lambda i: (i, 0)),
        out_shape=jax.ShapeDtypeStruct((NP, D_OUT), jnp.float32),
    )(p0p1, p0p1, g2, dinv, b2)


def kernel(x, edge_index, W1, b1, gamma, beta, W2, b2):
    src = edge_index[0].astype(jnp.int32)
    dst = edge_index[1].astype(jnp.int32)
    src1 = src.reshape(NW * (E_T // CH1), CH1)
    dst1 = dst.reshape(NW * (E_T // CH1), CH1)
    src2 = src.reshape(NW * (E_T // CH2), CH2)
    dst2 = dst.reshape(NW * (E_T // CH2), CH2)
    xp = jnp.pad(x, ((0, NP - N), (0, 0)))

    h1 = _matmul1(xp, W1)
    degp = _deg_kernel(dst)
    g1, dinv = _prescale(degp, h1)
    parts1 = _agg128(g1, src1, dst1)
    g2 = _bn_layer2(parts1, g1, dinv,
                    b1.reshape(1, D_HID), gamma.reshape(1, D_HID),
                    beta.reshape(1, D_HID), W2)
    parts2 = _agg64(g2, src2, dst2)
    out = _final(parts2, g2, dinv, b2.reshape(1, D_OUT))
    return out[:N]


# speedup vs baseline: 1.0285x; 1.0285x over previous
import functools

import jax
import jax.numpy as jnp
from jax import lax
from jax.experimental import pallas as pl
from jax.experimental.pallas import tpu as pltpu
from jax.experimental.pallas import tpu_sc as plsc

N = 10000
NP = 10240
E = 320000
D_IN = 128
D_HID = 128
D_OUT = 64

NC = 2
NS = 16
NW = NC * NS
E_T = E // NW
CH1 = 80
CH2 = 125
NBUF = 4

R = 512
GRID = NP // R

_mesh = plsc.VectorSubcoreMesh(
    core_axis_name="c", subcore_axis_name="s", num_cores=NC, num_subcores=NS)


@functools.partial(
    pl.kernel,
    out_type=jax.ShapeDtypeStruct((NW, NP), jnp.float32),
    mesh=_mesh,
    compiler_params=pltpu.CompilerParams(needs_layout_passes=False),
    scratch_types=[
        pltpu.VMEM((E_T,), jnp.int32),
        pltpu.VMEM((NP,), jnp.float32),
    ],
)
def _deg_kernel(dst_hbm, degp_hbm, idx_v, deg_v):
    c = lax.axis_index("c")
    s = lax.axis_index("s")
    wid = c * NS + s
    base = wid * E_T
    pltpu.sync_copy(dst_hbm.at[pl.ds(base, E_T)], idx_v)

    zeros16 = jnp.zeros((16,), jnp.float32)

    def zbody(i, _):
        deg_v[pl.ds(i * 16, 16)] = zeros16
        return 0

    lax.fori_loop(0, NP // 16, zbody, 0, unroll=4)

    ones16 = jnp.ones((16,), jnp.float32)

    def body(i, _):
        idx = idx_v[pl.ds(i * 16, 16)]
        plsc.addupdate_scatter(deg_v, [idx], ones16)
        return 0

    lax.fori_loop(0, E_T // 16, body, 0, unroll=4)
    pltpu.sync_copy(deg_v, degp_hbm.at[wid])


def _make_agg(D, CH):
    NCH = E_T // CH
    NFULL = (NCH // NBUF) * NBUF
    REM = NCH - NFULL
    assert REM in (0, 1)
    ZR = 80
    assert CH >= ZR and (NP // NS) % ZR == 0

    @functools.partial(
        pl.kernel,
        out_type=jax.ShapeDtypeStruct((NC, NP, D), jnp.float32),
        mesh=_mesh,
        compiler_params=pltpu.CompilerParams(use_tc_tiling_on_sc=False),
        scratch_types=[
            pltpu.VMEM((NBUF, CH), jnp.int32),
            pltpu.VMEM((NBUF, CH), jnp.int32),
            pltpu.VMEM((NBUF, CH, D), jnp.float32),
            pltpu.VMEM_SHARED((NP, D), jnp.float32),
        ] + [pltpu.SemaphoreType.DMA] * (4 * NBUF + 1),
    )
    def agg(g_hbm, src_hbm, dst_hbm, out_hbm, isrc_v, idst_v, rows_v,
            part_sh, *sems):
        gsem = sems[0:NBUF]
        ssem = sems[NBUF:2 * NBUF]
        issem = sems[2 * NBUF:3 * NBUF]
        idsem = sems[3 * NBUF:4 * NBUF]
        zsem = sems[4 * NBUF]
        c = lax.axis_index("c")
        s = lax.axis_index("s")
        wid = c * NS + s
        ibase = wid * NCH

        def issue_isrc(j, b):
            pltpu.async_copy(src_hbm.at[ibase + j], isrc_v.at[b], issem[b])

        def drain_isrc(b):
            pltpu.make_async_copy(src_hbm.at[0], isrc_v.at[b], issem[b]).wait()

        def issue_idst(j, b):
            pltpu.async_copy(dst_hbm.at[ibase + j], idst_v.at[b], idsem[b])

        def drain_idst(b):
            pltpu.make_async_copy(dst_hbm.at[0], idst_v.at[b], idsem[b]).wait()

        def issue_g(b):
            pltpu.async_copy(g_hbm.at[isrc_v.at[b]], rows_v.at[b], gsem[b])

        def drain_g(b):
            pltpu.make_async_copy(
                g_hbm.at[isrc_v.at[0]], rows_v.at[b], gsem[b]).wait()

        def issue_s(b):
            pltpu.async_copy(rows_v.at[b], part_sh.at[idst_v.at[b]], ssem[b],
                             add=True)

        def drain_s(b):
            pltpu.make_async_copy(
                rows_v.at[b], part_sh.at[idst_v.at[0]], ssem[b]).wait()

        for b in range(3):
            issue_isrc(b, b)
        for b in range(2):
            issue_idst(b, b)

        zeros16 = jnp.zeros((16,), jnp.float32)

        def zb(i, _):
            r = i // (D // 16)
            q = i % (D // 16)
            rows_v[0, r, pl.ds(q * 16, 16)] = zeros16
            return 0

        lax.fori_loop(0, ZR * (D // 16), zb, 0, unroll=4)

        rows_per = NP // NS
        nz = rows_per // ZR
        zsrc = rows_v.at[0, pl.ds(0, ZR)]
        for k in range(nz):
            pltpu.async_copy(zsrc, part_sh.at[pl.ds(s * rows_per + k * ZR, ZR)],
                             zsem)
        for k in range(nz):
            pltpu.make_async_copy(
                zsrc, part_sh.at[pl.ds(s * rows_per, ZR)], zsem).wait()
        plsc.subcore_barrier()

        drain_isrc(0)
        issue_g(0)
        drain_isrc(1)
        issue_g(1)

        def body(i, _):
            j0 = NBUF * i
            for b in range(NBUF):
                j = j0 + b
                b2 = (b + 2) % NBUF
                b3 = (b + 3) % NBUF
                drain_g(b)
                drain_idst(b)
                issue_s(b)
                if b >= 2:
                    drain_s(b2)
                else:
                    @pl.when(i > 0)
                    def _():
                        drain_s(b2)

                @pl.when(j + 2 < NCH)
                def _():
                    drain_isrc(b2)
                    issue_g(b2)
                    issue_idst(j + 2, b2)

                @pl.when(j + 3 < NCH)
                def _():
                    issue_isrc(j + 3, b3)

            return 0

        lax.fori_loop(0, NFULL // NBUF, body, 0)

        if REM:
            drain_g(0)
            drain_idst(0)
            issue_s(0)
            drain_s((NCH - 3) % NBUF)
            drain_s((NCH - 2) % NBUF)
            drain_s((NCH - 1) % NBUF)
        else:
            drain_s((NCH - 2) % NBUF)
            drain_s((NCH - 1) % NBUF)
        plsc.subcore_barrier()

        pltpu.sync_copy(part_sh.at[pl.ds(s * rows_per, rows_per)],
                        out_hbm.at[c, pl.ds(s * rows_per, rows_per)])

    return agg


_agg128 = _make_agg(D_HID, CH1)
_agg64 = _make_agg(D_OUT, CH2)


def _k2_body(degp_ref, x_ref, w1_ref, g1_ref, dinv_ref):
    deg = degp_ref[...]
    ones = jnp.ones((NW, 1), jnp.float32)
    degsum = lax.dot_general(deg, ones, (((0,), (0,)), ((), ())),
                             preferred_element_type=jnp.float32) + 1.0
    dinv = lax.rsqrt(degsum)
    h = jnp.dot(x_ref[...], w1_ref[...], preferred_element_type=jnp.float32)
    g1_ref[...] = h * dinv
    dinv_ref[...] = dinv


def _prescale(degp, x, W1):
    return pl.pallas_call(
        _k2_body,
        grid=(GRID,),
        in_specs=[
            pl.BlockSpec((NW, R), lambda i: (0, i)),
            pl.BlockSpec((R, D_IN), lambda i: (i, 0)),
            pl.BlockSpec((D_IN, D_HID), lambda i: (0, 0)),
        ],
        out_specs=[
            pl.BlockSpec((R, D_HID), lambda i: (i, 0)),
            pl.BlockSpec((R, 1), lambda i: (i, 0)),
        ],
        out_shape=[
            jax.ShapeDtypeStruct((NP, D_HID), jnp.float32),
            jax.ShapeDtypeStruct((NP, 1), jnp.float32),
        ],
    )(degp, x, W1)


def _k45_body(p0_ref, p1_ref, g1_ref, dinv_ref, b1_ref, gamma_ref, beta_ref,
              w2_ref, g2_ref, out1_sc, sum_sc, sq_sc, ss_sc):
    i = pl.program_id(0)

    @pl.when(i < GRID)
    def _():
        out1 = (dinv_ref[...] * (p0_ref[0] + p1_ref[0] + g1_ref[...])
                + b1_ref[...])
        out1_sc[pl.ds(i * R, R), :] = out1

        @pl.when(i == 0)
        def _():
            sum_sc[...] = jnp.zeros_like(sum_sc)
            sq_sc[...] = jnp.zeros_like(sq_sc)

        rows = lax.broadcasted_iota(jnp.int32, (R, 1), 0) + i * R
        out1m = jnp.where(rows < N, out1, 0.0)
        sum_sc[...] += jnp.sum(out1m, axis=0, keepdims=True)
        sq_sc[...] += jnp.sum(out1m * out1m, axis=0, keepdims=True)

        @pl.when(i == GRID - 1)
        def _():
            mean = sum_sc[...] / N
            var = sq_sc[...] / N - mean * mean
            scale = gamma_ref[...] * lax.rsqrt(var + 1e-5)
            shift = beta_ref[...] - mean * scale
            ss_sc[...] = jnp.concatenate([scale, shift], axis=0)

    @pl.when(i >= GRID)
    def _():
        k = i - GRID
        out1 = out1_sc[pl.ds(k * R, R), :]
        a = jax.nn.relu(out1 * ss_sc[0:1, :] + ss_sc[1:2, :])
        h2 = jnp.dot(a, w2_ref[...], preferred_element_type=jnp.float32)
        g2_ref[...] = h2 * dinv_ref[...]


def _bn_layer2(p0p1, g1, dinv, b1, gamma, beta, W2):
    return pl.pallas_call(
        _k45_body,
        grid=(2 * GRID,),
        in_specs=[
            pl.BlockSpec((1, R, D_HID),
                         lambda i: (0, jnp.minimum(i, GRID - 1), 0)),
            pl.BlockSpec((1, R, D_HID),
                         lambda i: (1, jnp.minimum(i, GRID - 1), 0)),
            pl.BlockSpec((R, D_HID), lambda i: (jnp.minimum(i, GRID - 1), 0)),
            pl.BlockSpec((R, 1), lambda i: (i % GRID, 0)),
            pl.BlockSpec((1, D_HID), lambda i: (0, 0)),
            pl.BlockSpec((1, D_HID), lambda i: (0, 0)),
            pl.BlockSpec((1, D_HID), lambda i: (0, 0)),
            pl.BlockSpec((D_HID, D_OUT), lambda i: (0, 0)),
        ],
        out_specs=pl.BlockSpec((R, D_OUT),
                               lambda i: (jnp.maximum(i - GRID, 0), 0)),
        out_shape=jax.ShapeDtypeStruct((NP, D_OUT), jnp.float32),
        scratch_shapes=[
            pltpu.VMEM((NP, D_HID), jnp.float32),
            pltpu.VMEM((1, D_HID), jnp.float32),
            pltpu.VMEM((1, D_HID), jnp.float32),
            pltpu.VMEM((2, D_HID), jnp.float32),
        ],
    )(p0p1, p0p1, g1, dinv, b1, gamma, beta, W2)


R7 = 400


def _k7_body(p0_ref, p1_ref, g2_ref, dinv_ref, b2_ref, out_ref):
    out_ref[...] = (dinv_ref[...] * (p0_ref[0] + p1_ref[0] + g2_ref[...])
                    + b2_ref[...])


def _final(parts, g2, dinv, b2):
    return pl.pallas_call(
        _k7_body,
        grid=(N // R7,),
        in_specs=[
            pl.BlockSpec((1, R7, D_OUT), lambda i: (0, i, 0)),
            pl.BlockSpec((1, R7, D_OUT), lambda i: (1, i, 0)),
            pl.BlockSpec((R7, D_OUT), lambda i: (i, 0)),
            pl.BlockSpec((R7, 1), lambda i: (i, 0)),
            pl.BlockSpec((1, D_OUT), lambda i: (0, 0)),
        ],
        out_specs=pl.BlockSpec((R7, D_OUT), lambda i: (i, 0)),
        out_shape=jax.ShapeDtypeStruct((N, D_OUT), jnp.float32),
    )(parts, parts, g2, dinv, b2)


def kernel(x, edge_index, W1, b1, gamma, beta, W2, b2):
    src = edge_index[0].astype(jnp.int32)
    dst = edge_index[1].astype(jnp.int32)
    src1 = src.reshape(NW * (E_T // CH1), CH1)
    dst1 = dst.reshape(NW * (E_T // CH1), CH1)
    src2 = src.reshape(NW * (E_T // CH2), CH2)
    dst2 = dst.reshape(NW * (E_T // CH2), CH2)
    xp = jnp.pad(x, ((0, NP - N), (0, 0)))

    degp = _deg_kernel(dst)
    g1, dinv = _prescale(degp, xp, W1)
    parts1 = _agg128(g1, src1, dst1)
    g2 = _bn_layer2(parts1, g1, dinv,
                    b1.reshape(1, D_HID), gamma.reshape(1, D_HID),
                    beta.reshape(1, D_HID), W2)
    parts2 = _agg64(g2, src2, dst2)
    return _final(parts2, g2, dinv, b2.reshape(1, D_OUT))


# speedup vs baseline: 1.1058x; 1.0751x over previous
import functools

import jax
import jax.numpy as jnp
from jax import lax
from jax.experimental import pallas as pl
from jax.experimental.pallas import tpu as pltpu
from jax.experimental.pallas import tpu_sc as plsc

N = 10000
NP = 10240
E = 320000
D_IN = 128
D_HID = 128
D_OUT = 64

NC = 2
NS = 16
NW = NC * NS
E_T = E // NW
CH1 = 80
CH2 = 80
NBUF = 4

R = 2048
GRID = NP // R

_mesh = plsc.VectorSubcoreMesh(
    core_axis_name="c", subcore_axis_name="s", num_cores=NC, num_subcores=NS)


@functools.partial(
    pl.kernel,
    out_type=jax.ShapeDtypeStruct((NW, NP), jnp.float32),
    mesh=_mesh,
    compiler_params=pltpu.CompilerParams(needs_layout_passes=False),
    scratch_types=[
        pltpu.VMEM((E_T,), jnp.int32),
        pltpu.VMEM((NP,), jnp.float32),
    ],
)
def _deg_kernel(dst_hbm, degp_hbm, idx_v, deg_v):
    c = lax.axis_index("c")
    s = lax.axis_index("s")
    wid = c * NS + s
    base = wid * E_T
    pltpu.sync_copy(dst_hbm.at[pl.ds(base, E_T)], idx_v)

    zeros16 = jnp.zeros((16,), jnp.float32)

    def zbody(i, _):
        deg_v[pl.ds(i * 16, 16)] = zeros16
        return 0

    lax.fori_loop(0, NP // 16, zbody, 0, unroll=4)

    ones16 = jnp.ones((16,), jnp.float32)

    def body(i, _):
        idx = idx_v[pl.ds(i * 16, 16)]
        plsc.addupdate_scatter(deg_v, [idx], ones16)
        return 0

    lax.fori_loop(0, E_T // 16, body, 0, unroll=4)
    pltpu.sync_copy(deg_v, degp_hbm.at[wid])


def _make_agg(D, CH):
    NCH = E_T // CH
    NFULL = (NCH // NBUF) * NBUF
    REM = NCH - NFULL
    assert REM in (0, 1)
    ZR = 80
    assert CH >= ZR and (NP // NS) % ZR == 0

    @functools.partial(
        pl.kernel,
        out_type=jax.ShapeDtypeStruct((NC, NP, D), jnp.float32),
        mesh=_mesh,
        compiler_params=pltpu.CompilerParams(use_tc_tiling_on_sc=False),
        scratch_types=[
            pltpu.VMEM((NBUF, CH), jnp.int32),
            pltpu.VMEM((NBUF, CH), jnp.int32),
            pltpu.VMEM((NBUF, CH, D), jnp.float32),
            pltpu.VMEM_SHARED((NP, D), jnp.float32),
        ] + [pltpu.SemaphoreType.DMA] * (4 * NBUF + 1),
    )
    def agg(g_hbm, src_hbm, dst_hbm, out_hbm, isrc_v, idst_v, rows_v,
            part_sh, *sems):
        gsem = sems[0:NBUF]
        ssem = sems[NBUF:2 * NBUF]
        issem = sems[2 * NBUF:3 * NBUF]
        idsem = sems[3 * NBUF:4 * NBUF]
        zsem = sems[4 * NBUF]
        c = lax.axis_index("c")
        s = lax.axis_index("s")
        wid = c * NS + s
        ebase = wid * E_T

        def issue_isrc(j, b):
            pltpu.async_copy(src_hbm.at[pl.ds(ebase + j * CH, CH)],
                             isrc_v.at[b], issem[b])

        def drain_isrc(b):
            pltpu.make_async_copy(src_hbm.at[pl.ds(0, CH)], isrc_v.at[b],
                                  issem[b]).wait()

        def issue_idst(j, b):
            pltpu.async_copy(dst_hbm.at[pl.ds(ebase + j * CH, CH)],
                             idst_v.at[b], idsem[b])

        def drain_idst(b):
            pltpu.make_async_copy(dst_hbm.at[pl.ds(0, CH)], idst_v.at[b],
                                  idsem[b]).wait()

        def issue_g(b):
            pltpu.async_copy(g_hbm.at[isrc_v.at[b]], rows_v.at[b], gsem[b])

        def drain_g(b):
            pltpu.make_async_copy(
                g_hbm.at[isrc_v.at[0]], rows_v.at[b], gsem[b]).wait()

        def issue_s(b):
            pltpu.async_copy(rows_v.at[b], part_sh.at[idst_v.at[b]], ssem[b],
                             add=True)

        def drain_s(b):
            pltpu.make_async_copy(
                rows_v.at[b], part_sh.at[idst_v.at[0]], ssem[b]).wait()

        for b in range(3):
            issue_isrc(b, b)
        for b in range(2):
            issue_idst(b, b)

        zeros16 = jnp.zeros((16,), jnp.float32)

        def zb(i, _):
            r = i // (D // 16)
            q = i % (D // 16)
            rows_v[0, r, pl.ds(q * 16, 16)] = zeros16
            return 0

        lax.fori_loop(0, ZR * (D // 16), zb, 0, unroll=4)

        rows_per = NP // NS
        nz = rows_per // ZR
        zsrc = rows_v.at[0, pl.ds(0, ZR)]
        for k in range(nz):
            pltpu.async_copy(zsrc, part_sh.at[pl.ds(s * rows_per + k * ZR, ZR)],
                             zsem)
        for k in range(nz):
            pltpu.make_async_copy(
                zsrc, part_sh.at[pl.ds(s * rows_per, ZR)], zsem).wait()
        plsc.subcore_barrier()

        drain_isrc(0)
        issue_g(0)
        drain_isrc(1)
        issue_g(1)

        def body(i, _):
            j0 = NBUF * i
            for b in range(NBUF):
                j = j0 + b
                b2 = (b + 2) % NBUF
                b3 = (b + 3) % NBUF
                drain_g(b)
                drain_idst(b)
                issue_s(b)
                if b >= 2:
                    drain_s(b2)
                else:
                    @pl.when(i > 0)
                    def _():
                        drain_s(b2)

                @pl.when(j + 2 < NCH)
                def _():
                    drain_isrc(b2)
                    issue_g(b2)
                    issue_idst(j + 2, b2)

                @pl.when(j + 3 < NCH)
                def _():
                    issue_isrc(j + 3, b3)

            return 0

        lax.fori_loop(0, NFULL // NBUF, body, 0)

        if REM:
            drain_g(0)
            drain_idst(0)
            issue_s(0)
            drain_s((NCH - 3) % NBUF)
            drain_s((NCH - 2) % NBUF)
            drain_s((NCH - 1) % NBUF)
        else:
            drain_s((NCH - 2) % NBUF)
            drain_s((NCH - 1) % NBUF)
        plsc.subcore_barrier()

        pltpu.sync_copy(part_sh.at[pl.ds(s * rows_per, rows_per)],
                        out_hbm.at[c, pl.ds(s * rows_per, rows_per)])

    return agg


_agg128 = _make_agg(D_HID, CH1)
_agg64 = _make_agg(D_OUT, CH2)


def _dinv_col(degp_blk):
    ones = jnp.ones((NW, 1), jnp.float32)
    degsum = lax.dot_general(degp_blk, ones, (((0,), (0,)), ((), ())),
                             preferred_element_type=jnp.float32) + 1.0
    return lax.rsqrt(degsum)


def _k2_body(degp_ref, x_ref, w1_ref, g1_ref):
    dinv = _dinv_col(degp_ref[...])
    h = jnp.dot(x_ref[...], w1_ref[...], preferred_element_type=jnp.float32)
    g1_ref[...] = h * dinv


def _prescale(degp, x, W1):
    return pl.pallas_call(
        _k2_body,
        grid=(GRID,),
        in_specs=[
            pl.BlockSpec((NW, R), lambda i: (0, i)),
            pl.BlockSpec((R, D_IN), lambda i: (i, 0)),
            pl.BlockSpec((D_IN, D_HID), lambda i: (0, 0)),
        ],
        out_specs=pl.BlockSpec((R, D_HID), lambda i: (i, 0)),
        out_shape=jax.ShapeDtypeStruct((NP, D_HID), jnp.float32),
    )(degp, x, W1)


def _k45_body(p0_ref, p1_ref, g1_ref, degp_ref, b1_ref, gamma_ref, beta_ref,
              w2_ref, g2_ref, out1_sc, sum_sc, sq_sc, ss_sc):
    i = pl.program_id(0)

    @pl.when(i < GRID)
    def _():
        dinv = _dinv_col(degp_ref[...])
        out1 = (dinv * (p0_ref[0] + p1_ref[0] + g1_ref[...])
                + b1_ref[...])
        out1_sc[pl.ds(i * R, R), :] = out1

        @pl.when(i == 0)
        def _():
            sum_sc[...] = jnp.zeros_like(sum_sc)
            sq_sc[...] = jnp.zeros_like(sq_sc)

        rows = lax.broadcasted_iota(jnp.int32, (R, 1), 0) + i * R
        out1m = jnp.where(rows < N, out1, 0.0)
        sum_sc[...] += jnp.sum(out1m, axis=0, keepdims=True)
        sq_sc[...] += jnp.sum(out1m * out1m, axis=0, keepdims=True)

        @pl.when(i == GRID - 1)
        def _():
            mean = sum_sc[...] / N
            var = sq_sc[...] / N - mean * mean
            scale = gamma_ref[...] * lax.rsqrt(var + 1e-5)
            shift = beta_ref[...] - mean * scale
            ss_sc[...] = jnp.concatenate([scale, shift], axis=0)

    @pl.when(i >= GRID)
    def _():
        k = i - GRID
        out1 = out1_sc[pl.ds(k * R, R), :]
        a = jax.nn.relu(out1 * ss_sc[0:1, :] + ss_sc[1:2, :])
        h2 = jnp.dot(a, w2_ref[...], preferred_element_type=jnp.float32)
        g2_ref[...] = h2 * _dinv_col(degp_ref[...])


def _bn_layer2(p0p1, g1, degp, b1, gamma, beta, W2):
    return pl.pallas_call(
        _k45_body,
        grid=(2 * GRID,),
        in_specs=[
            pl.BlockSpec((1, R, D_HID),
                         lambda i: (0, jnp.minimum(i, GRID - 1), 0)),
            pl.BlockSpec((1, R, D_HID),
                         lambda i: (1, jnp.minimum(i, GRID - 1), 0)),
            pl.BlockSpec((R, D_HID), lambda i: (jnp.minimum(i, GRID - 1), 0)),
            pl.BlockSpec((NW, R), lambda i: (0, i % GRID)),
            pl.BlockSpec((1, D_HID), lambda i: (0, 0)),
            pl.BlockSpec((1, D_HID), lambda i: (0, 0)),
            pl.BlockSpec((1, D_HID), lambda i: (0, 0)),
            pl.BlockSpec((D_HID, D_OUT), lambda i: (0, 0)),
        ],
        out_specs=pl.BlockSpec((R, D_OUT),
                               lambda i: (jnp.maximum(i - GRID, 0), 0)),
        out_shape=jax.ShapeDtypeStruct((NP, D_OUT), jnp.float32),
        scratch_shapes=[
            pltpu.VMEM((NP, D_HID), jnp.float32),
            pltpu.VMEM((1, D_HID), jnp.float32),
            pltpu.VMEM((1, D_HID), jnp.float32),
            pltpu.VMEM((2, D_HID), jnp.float32),
        ],
    )(p0p1, p0p1, g1, degp, b1, gamma, beta, W2)


def _k7_body(p0_ref, p1_ref, g2_ref, degp_ref, b2_ref, out_ref):
    dinv = _dinv_col(degp_ref[...])
    out_ref[...] = (dinv * (p0_ref[0] + p1_ref[0] + g2_ref[...])
                    + b2_ref[...])


def _final(parts, g2, degp, b2):
    return pl.pallas_call(
        _k7_body,
        grid=(GRID,),
        in_specs=[
            pl.BlockSpec((1, R, D_OUT), lambda i: (0, i, 0)),
            pl.BlockSpec((1, R, D_OUT), lambda i: (1, i, 0)),
            pl.BlockSpec((R, D_OUT), lambda i: (i, 0)),
            pl.BlockSpec((NW, R), lambda i: (0, i)),
            pl.BlockSpec((1, D_OUT), lambda i: (0, 0)),
        ],
        out_specs=pl.BlockSpec((R, D_OUT), lambda i: (i, 0)),
        out_shape=jax.ShapeDtypeStruct((NP, D_OUT), jnp.float32),
    )(parts, parts, g2, degp, b2)


def kernel(x, edge_index, W1, b1, gamma, beta, W2, b2):
    src = edge_index[0].astype(jnp.int32)
    dst = edge_index[1].astype(jnp.int32)
    xp = jnp.pad(x, ((0, NP - N), (0, 0)))

    degp = _deg_kernel(dst)
    g1 = _prescale(degp, xp, W1)
    parts1 = _agg128(g1, src, dst)
    g2 = _bn_layer2(parts1, g1, degp,
                    b1.reshape(1, D_HID), gamma.reshape(1, D_HID),
                    beta.reshape(1, D_HID), W2)
    parts2 = _agg64(g2, src, dst)
    return _final(parts2, g2, degp, b2.reshape(1, D_OUT))[:N]


# speedup vs baseline: 1.1650x; 1.0536x over previous
import functools

import jax
import jax.numpy as jnp
from jax import lax
from jax.experimental import pallas as pl
from jax.experimental.pallas import tpu as pltpu
from jax.experimental.pallas import tpu_sc as plsc

N = 10000
NP = 10240
E = 320000
D_IN = 128
D_HID = 128
D_OUT = 64

NC = 2
NS = 16
NW = NC * NS
E_T = E // NW
CH1 = 80
CH2 = 125
NBUF = 4

R = 2048
GRID = NP // R

_mesh = plsc.VectorSubcoreMesh(
    core_axis_name="c", subcore_axis_name="s", num_cores=NC, num_subcores=NS)


@functools.partial(
    pl.kernel,
    out_type=jax.ShapeDtypeStruct((NW, NP), jnp.float32),
    mesh=_mesh,
    compiler_params=pltpu.CompilerParams(needs_layout_passes=False),
    scratch_types=[
        pltpu.VMEM((E_T,), jnp.int32),
        pltpu.VMEM((NP,), jnp.float32),
    ],
)
def _deg_kernel(dst_hbm, degp_hbm, idx_v, deg_v):
    c = lax.axis_index("c")
    s = lax.axis_index("s")
    wid = c * NS + s
    base = wid * E_T
    pltpu.sync_copy(dst_hbm.at[pl.ds(base, E_T)], idx_v)

    zeros16 = jnp.zeros((16,), jnp.float32)

    def zbody(i, _):
        deg_v[pl.ds(i * 16, 16)] = zeros16
        return 0

    lax.fori_loop(0, NP // 16, zbody, 0, unroll=4)

    ones16 = jnp.ones((16,), jnp.float32)

    def body(i, _):
        idx = idx_v[pl.ds(i * 16, 16)]
        plsc.addupdate_scatter(deg_v, [idx], ones16)
        return 0

    lax.fori_loop(0, E_T // 16, body, 0, unroll=4)
    pltpu.sync_copy(deg_v, degp_hbm.at[wid])


def _make_agg(D, CH, idx2d=False):
    NCH = E_T // CH
    NFULL = (NCH // NBUF) * NBUF
    REM = NCH - NFULL
    assert REM in (0, 1)
    ZR = 80
    assert CH >= ZR and (NP // NS) % ZR == 0

    @functools.partial(
        pl.kernel,
        out_type=jax.ShapeDtypeStruct((NC, NP, D), jnp.float32),
        mesh=_mesh,
        compiler_params=pltpu.CompilerParams(use_tc_tiling_on_sc=False),
        scratch_types=[
            pltpu.VMEM((NBUF, CH), jnp.int32),
            pltpu.VMEM((NBUF, CH), jnp.int32),
            pltpu.VMEM((NBUF, CH, D), jnp.float32),
            pltpu.VMEM_SHARED((NP, D), jnp.float32),
        ] + [pltpu.SemaphoreType.DMA] * (4 * NBUF + 1),
    )
    def agg(g_hbm, src_hbm, dst_hbm, out_hbm, isrc_v, idst_v, rows_v,
            part_sh, *sems):
        gsem = sems[0:NBUF]
        ssem = sems[NBUF:2 * NBUF]
        issem = sems[2 * NBUF:3 * NBUF]
        idsem = sems[3 * NBUF:4 * NBUF]
        zsem = sems[4 * NBUF]
        c = lax.axis_index("c")
        s = lax.axis_index("s")
        wid = c * NS + s
        if idx2d:
            ibase = wid * NCH

            def _islice(ref, j):
                return ref.at[ibase + j]

            def _idummy(ref):
                return ref.at[0]
        else:
            ebase = wid * E_T

            def _islice(ref, j):
                return ref.at[pl.ds(ebase + j * CH, CH)]

            def _idummy(ref):
                return ref.at[pl.ds(0, CH)]

        def issue_isrc(j, b):
            pltpu.async_copy(_islice(src_hbm, j), isrc_v.at[b], issem[b])

        def drain_isrc(b):
            pltpu.make_async_copy(_idummy(src_hbm), isrc_v.at[b],
                                  issem[b]).wait()

        def issue_idst(j, b):
            pltpu.async_copy(_islice(dst_hbm, j), idst_v.at[b], idsem[b])

        def drain_idst(b):
            pltpu.make_async_copy(_idummy(dst_hbm), idst_v.at[b],
                                  idsem[b]).wait()

        def issue_g(b):
            pltpu.async_copy(g_hbm.at[isrc_v.at[b]], rows_v.at[b], gsem[b])

        def drain_g(b):
            pltpu.make_async_copy(
                g_hbm.at[isrc_v.at[0]], rows_v.at[b], gsem[b]).wait()

        def issue_s(b):
            pltpu.async_copy(rows_v.at[b], part_sh.at[idst_v.at[b]], ssem[b],
                             add=True)

        def drain_s(b):
            pltpu.make_async_copy(
                rows_v.at[b], part_sh.at[idst_v.at[0]], ssem[b]).wait()

        for b in range(3):
            issue_isrc(b, b)
        for b in range(2):
            issue_idst(b, b)

        zeros16 = jnp.zeros((16,), jnp.float32)

        def zb(i, _):
            r = i // (D // 16)
            q = i % (D // 16)
            rows_v[0, r, pl.ds(q * 16, 16)] = zeros16
            return 0

        lax.fori_loop(0, ZR * (D // 16), zb, 0, unroll=4)

        rows_per = NP // NS
        nz = rows_per // ZR
        zsrc = rows_v.at[0, pl.ds(0, ZR)]
        for k in range(nz):
            pltpu.async_copy(zsrc, part_sh.at[pl.ds(s * rows_per + k * ZR, ZR)],
                             zsem)
        for k in range(nz):
            pltpu.make_async_copy(
                zsrc, part_sh.at[pl.ds(s * rows_per, ZR)], zsem).wait()
        plsc.subcore_barrier()

        drain_isrc(0)
        issue_g(0)
        drain_isrc(1)
        issue_g(1)

        def body(i, _):
            j0 = NBUF * i
            for b in range(NBUF):
                j = j0 + b
                b2 = (b + 2) % NBUF
                b3 = (b + 3) % NBUF
                drain_g(b)
                drain_idst(b)
                issue_s(b)
                if b >= 2:
                    drain_s(b2)
                else:
                    @pl.when(i > 0)
                    def _():
                        drain_s(b2)

                @pl.when(j + 2 < NCH)
                def _():
                    drain_isrc(b2)
                    issue_g(b2)
                    issue_idst(j + 2, b2)

                @pl.when(j + 3 < NCH)
                def _():
                    issue_isrc(j + 3, b3)

            return 0

        lax.fori_loop(0, NFULL // NBUF, body, 0)

        if REM:
            drain_g(0)
            drain_idst(0)
            issue_s(0)
            drain_s((NCH - 3) % NBUF)
            drain_s((NCH - 2) % NBUF)
            drain_s((NCH - 1) % NBUF)
        else:
            drain_s((NCH - 2) % NBUF)
            drain_s((NCH - 1) % NBUF)
        plsc.subcore_barrier()

        pltpu.sync_copy(part_sh.at[pl.ds(s * rows_per, rows_per)],
                        out_hbm.at[c, pl.ds(s * rows_per, rows_per)])

    return agg


_agg128 = _make_agg(D_HID, CH1)
_agg64 = _make_agg(D_OUT, CH2, idx2d=True)


def _dinv_col(degp_blk):
    ones = jnp.ones((NW, 1), jnp.float32)
    degsum = lax.dot_general(degp_blk, ones, (((0,), (0,)), ((), ())),
                             preferred_element_type=jnp.float32) + 1.0
    return lax.rsqrt(degsum)


def _k2_body(degp_ref, x_ref, w1_ref, g1_ref):
    dinv = _dinv_col(degp_ref[...])
    h = jnp.dot(x_ref[...], w1_ref[...], preferred_element_type=jnp.float32)
    g1_ref[...] = h * dinv


def _prescale(degp, x, W1):
    return pl.pallas_call(
        _k2_body,
        grid=(GRID,),
        in_specs=[
            pl.BlockSpec((NW, R), lambda i: (0, i)),
            pl.BlockSpec((R, D_IN), lambda i: (i, 0)),
            pl.BlockSpec((D_IN, D_HID), lambda i: (0, 0)),
        ],
        out_specs=pl.BlockSpec((R, D_HID), lambda i: (i, 0)),
        out_shape=jax.ShapeDtypeStruct((NP, D_HID), jnp.float32),
    )(degp, x, W1)


def _k45_body(p0_ref, p1_ref, g1_ref, degp_ref, b1_ref, gamma_ref, beta_ref,
              w2_ref, g2_ref, out1_sc, sum_sc, sq_sc, ss_sc):
    i = pl.program_id(0)

    @pl.when(i < GRID)
    def _():
        dinv = _dinv_col(degp_ref[...])
        out1 = (dinv * (p0_ref[0] + p1_ref[0] + g1_ref[...])
                + b1_ref[...])
        out1_sc[pl.ds(i * R, R), :] = out1

        @pl.when(i == 0)
        def _():
            sum_sc[...] = jnp.zeros_like(sum_sc)
            sq_sc[...] = jnp.zeros_like(sq_sc)

        rows = lax.broadcasted_iota(jnp.int32, (R, 1), 0) + i * R
        out1m = jnp.where(rows < N, out1, 0.0)
        sum_sc[...] += jnp.sum(out1m, axis=0, keepdims=True)
        sq_sc[...] += jnp.sum(out1m * out1m, axis=0, keepdims=True)

        @pl.when(i == GRID - 1)
        def _():
            mean = sum_sc[...] / N
            var = sq_sc[...] / N - mean * mean
            scale = gamma_ref[...] * lax.rsqrt(var + 1e-5)
            shift = beta_ref[...] - mean * scale
            ss_sc[...] = jnp.concatenate([scale, shift], axis=0)

    @pl.when(i >= GRID)
    def _():
        k = i - GRID
        out1 = out1_sc[pl.ds(k * R, R), :]
        a = jax.nn.relu(out1 * ss_sc[0:1, :] + ss_sc[1:2, :])
        h2 = jnp.dot(a, w2_ref[...], preferred_element_type=jnp.float32)
        g2_ref[...] = h2 * _dinv_col(degp_ref[...])


def _bn_layer2(p0p1, g1, degp, b1, gamma, beta, W2):
    return pl.pallas_call(
        _k45_body,
        grid=(2 * GRID,),
        in_specs=[
            pl.BlockSpec((1, R, D_HID),
                         lambda i: (0, jnp.minimum(i, GRID - 1), 0)),
            pl.BlockSpec((1, R, D_HID),
                         lambda i: (1, jnp.minimum(i, GRID - 1), 0)),
            pl.BlockSpec((R, D_HID), lambda i: (jnp.minimum(i, GRID - 1), 0)),
            pl.BlockSpec((NW, R), lambda i: (0, i % GRID)),
            pl.BlockSpec((1, D_HID), lambda i: (0, 0)),
            pl.BlockSpec((1, D_HID), lambda i: (0, 0)),
            pl.BlockSpec((1, D_HID), lambda i: (0, 0)),
            pl.BlockSpec((D_HID, D_OUT), lambda i: (0, 0)),
        ],
        out_specs=pl.BlockSpec((R, D_OUT),
                               lambda i: (jnp.maximum(i - GRID, 0), 0)),
        out_shape=jax.ShapeDtypeStruct((NP, D_OUT), jnp.float32),
        scratch_shapes=[
            pltpu.VMEM((NP, D_HID), jnp.float32),
            pltpu.VMEM((1, D_HID), jnp.float32),
            pltpu.VMEM((1, D_HID), jnp.float32),
            pltpu.VMEM((2, D_HID), jnp.float32),
        ],
    )(p0p1, p0p1, g1, degp, b1, gamma, beta, W2)


def _k7_body(p0_ref, p1_ref, g2_ref, degp_ref, b2_ref, out_ref):
    dinv = _dinv_col(degp_ref[...])
    out_ref[...] = (dinv * (p0_ref[0] + p1_ref[0] + g2_ref[...])
                    + b2_ref[...])


def _final(parts, g2, degp, b2):
    return pl.pallas_call(
        _k7_body,
        grid=(GRID,),
        in_specs=[
            pl.BlockSpec((1, R, D_OUT), lambda i: (0, i, 0)),
            pl.BlockSpec((1, R, D_OUT), lambda i: (1, i, 0)),
            pl.BlockSpec((R, D_OUT), lambda i: (i, 0)),
            pl.BlockSpec((NW, R), lambda i: (0, i)),
            pl.BlockSpec((1, D_OUT), lambda i: (0, 0)),
        ],
        out_specs=pl.BlockSpec((R, D_OUT), lambda i: (i, 0)),
        out_shape=jax.ShapeDtypeStruct((NP, D_OUT), jnp.float32),
    )(parts, parts, g2, degp, b2)


def kernel(x, edge_index, W1, b1, gamma, beta, W2, b2):
    xp = jnp.pad(x, ((0, NP - N), (0, 0)))

    src = edge_index[0].astype(jnp.int32)
    dst = edge_index[1].astype(jnp.int32)
    degp = _deg_kernel(dst)
    g1 = _prescale(degp, xp, W1)
    parts1 = _agg128(g1, src, dst)
    g2 = _bn_layer2(parts1, g1, degp,
                    b1.reshape(1, D_HID), gamma.reshape(1, D_HID),
                    beta.reshape(1, D_HID), W2)
    src2 = src.reshape(NW * (E_T // CH2), CH2)
    dst2 = dst.reshape(NW * (E_T // CH2), CH2)
    parts2 = _agg64(g2, src2, dst2)
    return _final(parts2, g2, degp, b2.reshape(1, D_OUT))[:N]
